# baseline (speedup 1.0000x reference)
import functools

import jax
import jax.numpy as jnp
from jax import lax
from jax.experimental import pallas as pl
from jax.experimental.pallas import tpu as pltpu
from jax.experimental.pallas import tpu_sc as plsc

_NC = 2
_NS = 16
_NW = _NC * _NS
_CHUNK = 128


def _prep_body(w_ref, lhs_ref):
    w = w_ref[...]
    w_sq = jnp.sum(w * w, axis=1, keepdims=True)
    lhs_ref[...] = jnp.concatenate([-2.0 * w, w_sq], axis=1)


def _vq_tc_body(h_ref, lhs_ref, idx_ref, loss_ref):
    i = pl.program_id(0)
    K = lhs_ref.shape[0]
    hb = h_ref[...]
    D1 = hb.shape[1]
    sq_mask = (jax.lax.broadcasted_iota(jnp.int32, (1, D1), 1)
               < D1 - 1).astype(jnp.float32)
    h_sq = jax.lax.dot_general(
        sq_mask, hb * hb, (((1,), (1,)), ((), ())),
        preferred_element_type=jnp.float32)
    m2w = jax.lax.dot_general(
        lhs_ref[...], hb, (((1,), (1,)), ((), ())),
        preferred_element_type=jnp.float32)
    dist = h_sq + m2w
    minval = jnp.min(dist, axis=0, keepdims=True)
    onehot = jnp.where(dist == minval, 1.0, 0.0)
    iota_f = jax.lax.broadcasted_iota(
        jnp.int32, (1, K), 1).astype(jnp.float32)
    idx_f = jax.lax.dot_general(
        iota_f, onehot, (((1,), (0,)), ((), ())),
        preferred_element_type=jnp.float32)
    idx_ref[0, 0, :] = jnp.minimum(idx_f[0], float(K - 1)).astype(jnp.int32)

    @pl.when(i == 0)
    def _():
        loss_ref[0, 0] = 0.0

    loss_ref[0, 0] += jnp.sum(minval)


def _tc_stage(h_flat, W, BN):
    N, D = h_flat.shape
    K = W.shape[0]
    grid = N // BN

    lhs_aug = pl.pallas_call(
        _prep_body,
        out_shape=jax.ShapeDtypeStruct((K, D + 1), jnp.float32),
    )(W)

    h_aug = jnp.concatenate(
        [h_flat, jnp.ones((N, 1), jnp.float32)], axis=1)

    idx3, loss_sum = pl.pallas_call(
        _vq_tc_body,
        grid=(grid,),
        in_specs=[
            pl.BlockSpec((BN, D + 1), lambda i: (i, 0)),
            pl.BlockSpec((K, D + 1), lambda i: (0, 0)),
        ],
        out_specs=[
            pl.BlockSpec((1, 1, BN), lambda i: (i, 0, 0)),
            pl.BlockSpec((1, 1), lambda i: (0, 0), memory_space=pltpu.SMEM),
        ],
        out_shape=[
            jax.ShapeDtypeStruct((grid, 1, BN), jnp.int32),
            jax.ShapeDtypeStruct((1, 1), jnp.float32),
        ],
        compiler_params=pltpu.CompilerParams(
            dimension_semantics=("arbitrary",)),
    )(h_aug, lhs_aug)
    return idx3.reshape(N), loss_sum


def _make_sc_gather(N, K, D):
    b_per_w = N // _NW
    n_chunks = b_per_w // _CHUNK
    mesh = plsc.VectorSubcoreMesh(core_axis_name="c", subcore_axis_name="s")

    @functools.partial(
        pl.kernel,
        mesh=mesh,
        out_type=jax.ShapeDtypeStruct((N, D), jnp.float32),
        compiler_params=pltpu.CompilerParams(use_tc_tiling_on_sc=False),
        scratch_types=[
            pltpu.VMEM((b_per_w,), jnp.int32),
            pltpu.VMEM((b_per_w, D), jnp.float32),
            pltpu.SemaphoreType.DMA,
        ],
    )
    def gather_kernel(idx_hbm, table_hbm, out_hbm, idx_v, rows_v, sem):
        wid = lax.axis_index("s") * _NC + lax.axis_index("c")
        base = wid * b_per_w
        pltpu.sync_copy(idx_hbm.at[pl.ds(base, b_per_w)], idx_v)
        copies = []
        for c in range(n_chunks):
            copies.append(pltpu.async_copy(
                table_hbm.at[idx_v.at[pl.ds(c * _CHUNK, _CHUNK)]],
                rows_v.at[pl.ds(c * _CHUNK, _CHUNK)],
                sem))
        for cp in copies:
            cp.wait()
        pltpu.sync_copy(rows_v, out_hbm.at[pl.ds(base, b_per_w)])

    return gather_kernel


def kernel(h, W):
    N = h.shape[0] * h.shape[1]
    D = h.shape[2]
    K = W.shape[0]
    h_flat = h.reshape(N, D)

    idx, loss_sum = _tc_stage(h_flat, W, BN=512)
    q = _make_sc_gather(N, K, D)(idx, W)

    mse = loss_sum[0, 0] / jnp.float32(N * D)
    commitment_loss = jnp.float32(0.25) * mse
    codebook_loss = mse
    return q.reshape(h.shape), commitment_loss, codebook_loss


# speedup vs baseline: 1.1028x; 1.1028x over previous
import functools

import jax
import jax.numpy as jnp
from jax import lax
from jax.experimental import pallas as pl
from jax.experimental.pallas import tpu as pltpu
from jax.experimental.pallas import tpu_sc as plsc

_NC = 2
_NS = 16
_NW = _NC * _NS
_CHUNK = 128


def _prep_body(w_ref, lhs_ref):
    w = w_ref[...]
    w_sq = jnp.sum(w * w, axis=1, keepdims=True)
    lhs_ref[...] = jnp.concatenate([-2.0 * w, w_sq], axis=1)


def _vq_tc_body(h_ref, lhs_ref, idx_ref, loss_ref):
    i = pl.program_id(0)
    K = lhs_ref.shape[0]
    hb = h_ref[...]
    BN, D = hb.shape
    h_sq = jax.lax.dot_general(
        jnp.ones((1, D), jnp.float32), hb * hb,
        (((1,), (1,)), ((), ())),
        preferred_element_type=jnp.float32)
    rhs_aug = jnp.concatenate(
        [hb, jnp.ones((BN, 1), jnp.float32)], axis=1)
    m2w = jax.lax.dot_general(
        lhs_ref[...], rhs_aug, (((1,), (1,)), ((), ())),
        preferred_element_type=jnp.float32)
    dist = h_sq + m2w
    minval = jnp.min(dist, axis=0, keepdims=True)
    iota_f = jax.lax.broadcasted_iota(
        jnp.int32, dist.shape, 0).astype(jnp.float32)
    idx_f = jnp.min(jnp.where(dist == minval, iota_f, float(K)), axis=0)
    idx_ref[0, 0, :] = idx_f.astype(jnp.int32)

    @pl.when(i == 0)
    def _():
        loss_ref[0, 0] = 0.0

    loss_ref[0, 0] += jnp.sum(minval)


def _tc_stage(h_flat, W, BN):
    N, D = h_flat.shape
    K = W.shape[0]
    grid = N // BN

    lhs_aug = pl.pallas_call(
        _prep_body,
        out_shape=jax.ShapeDtypeStruct((K, D + 1), jnp.float32),
    )(W)

    idx3, loss_sum = pl.pallas_call(
        _vq_tc_body,
        grid=(grid,),
        in_specs=[
            pl.BlockSpec((BN, D), lambda i: (i, 0)),
            pl.BlockSpec((K, D + 1), lambda i: (0, 0)),
        ],
        out_specs=[
            pl.BlockSpec((1, 1, BN), lambda i: (i, 0, 0)),
            pl.BlockSpec((1, 1), lambda i: (0, 0), memory_space=pltpu.SMEM),
        ],
        out_shape=[
            jax.ShapeDtypeStruct((grid, 1, BN), jnp.int32),
            jax.ShapeDtypeStruct((1, 1), jnp.float32),
        ],
        compiler_params=pltpu.CompilerParams(
            dimension_semantics=("arbitrary",)),
    )(h_flat, lhs_aug)
    return idx3.reshape(N), loss_sum


def _make_sc_gather(N, K, D):
    b_per_w = N // _NW
    n_chunks = b_per_w // _CHUNK
    mesh = plsc.VectorSubcoreMesh(core_axis_name="c", subcore_axis_name="s")

    @functools.partial(
        pl.kernel,
        mesh=mesh,
        out_type=jax.ShapeDtypeStruct((N, D), jnp.float32),
        compiler_params=pltpu.CompilerParams(use_tc_tiling_on_sc=False),
        scratch_types=[
            pltpu.VMEM((b_per_w,), jnp.int32),
            pltpu.VMEM((b_per_w, D), jnp.float32),
            pltpu.SemaphoreType.DMA,
        ],
    )
    def gather_kernel(idx_hbm, table_hbm, out_hbm, idx_v, rows_v, sem):
        wid = lax.axis_index("s") * _NC + lax.axis_index("c")
        base = wid * b_per_w
        pltpu.sync_copy(idx_hbm.at[pl.ds(base, b_per_w)], idx_v)
        copies = []
        for c in range(n_chunks):
            copies.append(pltpu.async_copy(
                table_hbm.at[idx_v.at[pl.ds(c * _CHUNK, _CHUNK)]],
                rows_v.at[pl.ds(c * _CHUNK, _CHUNK)],
                sem))
        for cp in copies:
            cp.wait()
        pltpu.sync_copy(rows_v, out_hbm.at[pl.ds(base, b_per_w)])

    return gather_kernel


def kernel(h, W):
    N = h.shape[0] * h.shape[1]
    D = h.shape[2]
    K = W.shape[0]
    h_flat = h.reshape(N, D)

    idx, loss_sum = _tc_stage(h_flat, W, BN=512)
    q = _make_sc_gather(N, K, D)(idx, W)

    mse = loss_sum[0, 0] / jnp.float32(N * D)
    commitment_loss = jnp.float32(0.25) * mse
    codebook_loss = mse
    return q.reshape(h.shape), commitment_loss, codebook_loss
